# baseline (speedup 1.0000x reference)
import functools

import jax
import jax.numpy as jnp
from jax import lax
from jax.experimental import pallas as pl
from jax.experimental.pallas import tpu as pltpu
from jax.experimental.pallas import tpu_sc as plsc

N = 8448
B = 128
K = 19
KP = K + 1
NC = 2
NS = 16
NW = NC * NS
NPT = N // NW
NCH = 6
CPT = NPT // NCH
IDXW = NCH * KP
LANES = 16
NM = B // 32

_SGN16 = -0x8000
_MAG16 = 0x7FFF


def _widen(bf16x32):
    return plsc.unpack(bf16x32, format=plsc.PackFormat.INTERLEAVED)


def _narrow(a, b):
    ua = (plsc.bitcast(a, jnp.uint32) + jnp.uint32(0x8000)) >> jnp.uint32(16)
    ub = (plsc.bitcast(b, jnp.uint32) + jnp.uint32(0x8000)) >> jnp.uint32(16)
    return plsc.bitcast(ua | (ub << jnp.uint32(16)), jnp.int32)


def _iter_body(final, sbf, prv, wllr, w3, idx, out, sout,
               idx_v, rows0_v, rows1_v, io_v, prv_v, sout_v, w_v,
               sem0, sem1):
    wid = lax.axis_index("s") * NC + lax.axis_index("c")
    base = wid * NPT
    pltpu.sync_copy(idx.at[wid], idx_v)

    bufs = ((rows0_v, sem0), (rows1_v, sem1))

    def issue(c, b):
        pltpu.async_copy(sbf.at[idx_v.at[c]], bufs[b][0], bufs[b][1])

    issue(0, 0)
    issue(1, 1)
    pltpu.sync_copy(w3.at[pl.ds(base * 3, NPT * 3 + LANES)], w_v)
    pltpu.sync_copy(wllr.at[pl.ds(base, NPT)], io_v)
    pltpu.sync_copy(prv.at[pl.ds(base, NPT)], prv_v)

    def compute(c, rows_v):
        def node_body(i, _):
            j = c * NCH + i
            wv = w_v[pl.ds(j * 3, LANES)]
            wck = wv[0]
            wr0 = wv[1]
            wr1 = wv[2]
            r0 = i * KP
            for m in range(NM):
                slw = pl.ds(m * LANES, LANES)
                v = plsc.bitcast(rows_v[r0, slw], jnp.bfloat16)
                neg = v < jnp.bfloat16(0.0)
                mag = jnp.abs(v)
                for k in range(1, K):
                    v = plsc.bitcast(rows_v[r0 + k, slw], jnp.bfloat16)
                    neg = neg != (v < jnp.bfloat16(0.0))
                    mag = jnp.minimum(mag, jnp.abs(v))
                chk = jnp.where(neg, -mag, mag)
                cev, cod = _widen(chk)
                oev, ood = _widen(
                    plsc.bitcast(rows_v[r0 + K, slw], jnp.bfloat16))
                res_pair = []
                for g, ch, ow in ((2 * m, cev, oev), (2 * m + 1, cod, ood)):
                    sl = pl.ds(g * LANES, LANES)
                    res = (io_v[j, sl] + wck * ch
                           + wr0 * ow + wr1 * prv_v[j, sl])
                    if final:
                        res = 1.0 / (1.0 + jnp.exp(-res))
                    io_v[j, sl] = res
                    res_pair.append(res)
                sout_v[j, slw] = _narrow(res_pair[0], res_pair[1])
            return _

        lax.fori_loop(0, NCH, node_body, None, unroll=3)

    def outer(c0, _):
        for b in range(2):
            c = c0 + b
            pltpu.make_async_copy(
                sbf.at[idx_v.at[c]], bufs[b][0], bufs[b][1]).wait()
            compute(c, bufs[b][0])

            @pl.when(c + 2 < CPT)
            def _issue_next():
                issue(c + 2, b)
        return _

    lax.fori_loop(0, CPT // 2, lambda s, x: outer(s * 2, x), None)
    pltpu.sync_copy(io_v, out.at[pl.ds(base, NPT)])
    pltpu.sync_copy(sout_v, sout.at[pl.ds(base, NPT)])


@functools.partial(jax.jit, static_argnames=("final",))
def _bp_iter(sbf, prv, wllr, w3, idx, final):
    mesh = plsc.VectorSubcoreMesh(core_axis_name="c", subcore_axis_name="s")
    return pl.kernel(
        functools.partial(_iter_body, final),
        out_type=(
            jax.ShapeDtypeStruct((N, B), jnp.float32),
            jax.ShapeDtypeStruct((N, B // 2), jnp.int32),
        ),
        mesh=mesh,
        compiler_params=pltpu.CompilerParams(
            needs_layout_passes=False, use_tc_tiling_on_sc=False),
        scratch_types=[
            pltpu.VMEM((CPT, IDXW), jnp.int32),
            pltpu.VMEM((IDXW, B // 2), jnp.int32),
            pltpu.VMEM((IDXW, B // 2), jnp.int32),
            pltpu.VMEM((NPT, B), jnp.float32),
            pltpu.VMEM((NPT, B), jnp.float32),
            pltpu.VMEM((NPT, B // 2), jnp.int32),
            pltpu.VMEM((NPT * 3 + LANES,), jnp.float32),
            pltpu.SemaphoreType.DMA,
            pltpu.SemaphoreType.DMA,
        ],
    )(sbf, prv, wllr, w3, idx)


def _shadow_init(x_f32):
    n = x_f32.shape[0]
    inter = (x_f32.astype(jnp.bfloat16)
             .reshape(n, NM, 2, LANES)
             .transpose(0, 1, 3, 2)
             .reshape(n, B // 2, 2))
    return lax.bitcast_convert_type(inter, jnp.int32)


def kernel(input_llr, check_index_tensor, var_index_tensor, w_ch, w_check,
           w_res):
    del var_index_tensor
    llr_t = input_llr.T
    wllr_t = (input_llr * w_ch[None, :]).T
    own = jnp.arange(N, dtype=jnp.int32)[:, None]
    idx = jnp.concatenate(
        [check_index_tensor.astype(jnp.int32), own], axis=1)
    idx = idx.reshape(NW, CPT, IDXW)
    zeros = jnp.zeros_like(w_res[0])
    sbf = _shadow_init(wllr_t)
    prv = wllr_t
    out = wllr_t
    for t in range(5):
        w3 = jnp.stack(
            [w_check, w_res[0], w_res[1] if t > 0 else zeros], axis=1)
        w3 = jnp.concatenate(
            [w3.reshape(-1), jnp.zeros((LANES,), jnp.float32)])
        wl = wllr_t + llr_t if t == 4 else wllr_t
        new, new_sbf = _bp_iter(sbf, prv, wl, w3, idx, final=(t == 4))
        prv, out, sbf = out, new, new_sbf
    return out.T


# speedup vs baseline: 1.0711x; 1.0711x over previous
import functools

import jax
import jax.numpy as jnp
from jax import lax
from jax.experimental import pallas as pl
from jax.experimental.pallas import tpu as pltpu
from jax.experimental.pallas import tpu_sc as plsc

N = 8448
B = 128
K = 19
KP = K + 1
NC = 2
NS = 16
NW = NC * NS
NPT = N // NW
NCH = 6
CPT = NPT // NCH
IDXW = NCH * KP
LANES = 16
NM = B // 32

_SGN16 = -0x8000
_MAG16 = 0x7FFF


def _widen(bf16x32):
    return plsc.unpack(bf16x32, format=plsc.PackFormat.INTERLEAVED)


def _narrow(a, b):
    ua = (plsc.bitcast(a, jnp.uint32) + jnp.uint32(0x8000)) >> jnp.uint32(16)
    ub = (plsc.bitcast(b, jnp.uint32) + jnp.uint32(0x8000)) >> jnp.uint32(16)
    return plsc.bitcast(ua | (ub << jnp.uint32(16)), jnp.int32)


def _iter_body(final, sbf, prv, wllr, w3, idx, out, sout,
               idx_v, rows0_v, rows1_v, io_v, prv_v, sout_v, w_v,
               sem0, sem1):
    wid = lax.axis_index("s") * NC + lax.axis_index("c")
    base = wid * NPT
    pltpu.sync_copy(idx.at[wid], idx_v)

    bufs = ((rows0_v, sem0), (rows1_v, sem1))

    def issue(c, b):
        pltpu.async_copy(sbf.at[idx_v.at[c]], bufs[b][0], bufs[b][1])

    issue(0, 0)
    issue(1, 1)
    pltpu.sync_copy(w3.at[pl.ds(base * 3, NPT * 3 + LANES)], w_v)
    pltpu.sync_copy(wllr.at[pl.ds(base, NPT)], io_v)
    pltpu.sync_copy(prv.at[pl.ds(base, NPT)], prv_v)

    def compute(c, rows_v):
        def node_body(i, _):
            j = c * NCH + i
            wv = w_v[pl.ds(j * 3, LANES)]
            wck = wv[0]
            wr0 = wv[1]
            wr1 = wv[2]
            r0 = i * KP
            for m in range(NM):
                slw = pl.ds(m * LANES, LANES)
                v = plsc.bitcast(rows_v[r0, slw], jnp.bfloat16)
                neg = v < jnp.bfloat16(0.0)
                mag = jnp.abs(v)
                for k in range(1, K):
                    v = plsc.bitcast(rows_v[r0 + k, slw], jnp.bfloat16)
                    neg = neg != (v < jnp.bfloat16(0.0))
                    mag = jnp.minimum(mag, jnp.abs(v))
                chk = jnp.where(neg, -mag, mag)
                cev, cod = _widen(chk)
                oev, ood = _widen(
                    plsc.bitcast(rows_v[r0 + K, slw], jnp.bfloat16))
                res_pair = []
                for g, ch, ow in ((2 * m, cev, oev), (2 * m + 1, cod, ood)):
                    sl = pl.ds(g * LANES, LANES)
                    res = (io_v[j, sl] + wck * ch
                           + wr0 * ow + wr1 * prv_v[j, sl])
                    if final:
                        res = 1.0 / (1.0 + jnp.exp(-res))
                    io_v[j, sl] = res
                    res_pair.append(res)
                sout_v[j, slw] = _narrow(res_pair[0], res_pair[1])
            return _

        lax.fori_loop(0, NCH, node_body, None, unroll=2)

    def outer(c0, _):
        for b in range(2):
            c = c0 + b
            pltpu.make_async_copy(
                sbf.at[idx_v.at[c]], bufs[b][0], bufs[b][1]).wait()
            compute(c, bufs[b][0])

            @pl.when(c + 2 < CPT)
            def _issue_next():
                issue(c + 2, b)
        return _

    lax.fori_loop(0, CPT // 2, lambda s, x: outer(s * 2, x), None)
    pltpu.async_copy(io_v, out.at[pl.ds(base, NPT)], sem0)
    pltpu.async_copy(sout_v, sout.at[pl.ds(base, NPT)], sem1)
    pltpu.make_async_copy(io_v, out.at[pl.ds(base, NPT)], sem0).wait()
    pltpu.make_async_copy(sout_v, sout.at[pl.ds(base, NPT)], sem1).wait()


@functools.partial(jax.jit, static_argnames=("final",))
def _bp_iter(sbf, prv, wllr, w3, idx, final):
    mesh = plsc.VectorSubcoreMesh(core_axis_name="c", subcore_axis_name="s")
    return pl.kernel(
        functools.partial(_iter_body, final),
        out_type=(
            jax.ShapeDtypeStruct((N, B), jnp.float32),
            jax.ShapeDtypeStruct((N, B // 2), jnp.int32),
        ),
        mesh=mesh,
        compiler_params=pltpu.CompilerParams(
            needs_layout_passes=False, use_tc_tiling_on_sc=False),
        scratch_types=[
            pltpu.VMEM((CPT, IDXW), jnp.int32),
            pltpu.VMEM((IDXW, B // 2), jnp.int32),
            pltpu.VMEM((IDXW, B // 2), jnp.int32),
            pltpu.VMEM((NPT, B), jnp.float32),
            pltpu.VMEM((NPT, B), jnp.float32),
            pltpu.VMEM((NPT, B // 2), jnp.int32),
            pltpu.VMEM((NPT * 3 + LANES,), jnp.float32),
            pltpu.SemaphoreType.DMA,
            pltpu.SemaphoreType.DMA,
        ],
    )(sbf, prv, wllr, w3, idx)


def _shadow_init(x_f32):
    n = x_f32.shape[0]
    inter = (x_f32.astype(jnp.bfloat16)
             .reshape(n, NM, 2, LANES)
             .transpose(0, 1, 3, 2)
             .reshape(n, B // 2, 2))
    return lax.bitcast_convert_type(inter, jnp.int32)


def kernel(input_llr, check_index_tensor, var_index_tensor, w_ch, w_check,
           w_res):
    del var_index_tensor
    llr_t = input_llr.T
    wllr_t = (input_llr * w_ch[None, :]).T
    own = jnp.arange(N, dtype=jnp.int32)[:, None]
    idx = jnp.concatenate(
        [check_index_tensor.astype(jnp.int32), own], axis=1)
    idx = idx.reshape(NW, CPT, IDXW)
    zeros = jnp.zeros_like(w_res[0])
    sbf = _shadow_init(wllr_t)
    prv = wllr_t
    out = wllr_t
    for t in range(5):
        w3 = jnp.stack(
            [w_check, w_res[0], w_res[1] if t > 0 else zeros], axis=1)
        w3 = jnp.concatenate(
            [w3.reshape(-1), jnp.zeros((LANES,), jnp.float32)])
        wl = wllr_t + llr_t if t == 4 else wllr_t
        new, new_sbf = _bp_iter(sbf, prv, wl, w3, idx, final=(t == 4))
        prv, out, sbf = out, new, new_sbf
    return out.T


# speedup vs baseline: 1.0805x; 1.0087x over previous
import functools

import jax
import jax.numpy as jnp
from jax import lax
from jax.experimental import pallas as pl
from jax.experimental.pallas import tpu as pltpu
from jax.experimental.pallas import tpu_sc as plsc

N = 8448
B = 128
K = 19
KP = K + 1
NC = 2
NS = 16
NW = NC * NS
NPT = N // NW
NCH = 6
CPT = NPT // NCH
IDXW = NCH * KP
LANES = 16
NM = B // 32

_SGN16 = -0x8000
_MAG16 = 0x7FFF


def _widen(bf16x32):
    return plsc.unpack(bf16x32, format=plsc.PackFormat.INTERLEAVED)


def _narrow(a, b):
    ua = (plsc.bitcast(a, jnp.uint32) + jnp.uint32(0x8000)) >> jnp.uint32(16)
    ub = (plsc.bitcast(b, jnp.uint32) + jnp.uint32(0x8000)) >> jnp.uint32(16)
    return plsc.bitcast(ua | (ub << jnp.uint32(16)), jnp.int32)


def _iter_body(final, sbf, prv, wllr, w3, idx, out, sout,
               idx_v, rows0_v, rows1_v, io_v, prv_v, sout_v, w_v,
               sem0, sem1, sem2, sem3, sem4):
    wid = lax.axis_index("s") * NC + lax.axis_index("c")
    base = wid * NPT
    pltpu.sync_copy(idx.at[wid], idx_v)

    bufs = ((rows0_v, sem0), (rows1_v, sem1))

    def issue(c, b):
        pltpu.async_copy(sbf.at[idx_v.at[c]], bufs[b][0], bufs[b][1])

    issue(0, 0)
    issue(1, 1)
    pltpu.async_copy(w3.at[pl.ds(base * 3, NPT * 3 + LANES)], w_v, sem2)
    pltpu.async_copy(wllr.at[pl.ds(base, NPT)], io_v, sem3)
    pltpu.async_copy(prv.at[pl.ds(base, NPT)], prv_v, sem4)
    pltpu.make_async_copy(
        w3.at[pl.ds(base * 3, NPT * 3 + LANES)], w_v, sem2).wait()
    pltpu.make_async_copy(wllr.at[pl.ds(base, NPT)], io_v, sem3).wait()
    pltpu.make_async_copy(prv.at[pl.ds(base, NPT)], prv_v, sem4).wait()

    def compute(c, rows_v):
        def node_body(i, _):
            j = c * NCH + i
            wv = w_v[pl.ds(j * 3, LANES)]
            wck = wv[0]
            wr0 = wv[1]
            wr1 = wv[2]
            r0 = i * KP
            for m in range(NM):
                slw = pl.ds(m * LANES, LANES)
                v = plsc.bitcast(rows_v[r0, slw], jnp.bfloat16)
                neg = v < jnp.bfloat16(0.0)
                mag = jnp.abs(v)
                for k in range(1, K):
                    v = plsc.bitcast(rows_v[r0 + k, slw], jnp.bfloat16)
                    neg = neg != (v < jnp.bfloat16(0.0))
                    mag = jnp.minimum(mag, jnp.abs(v))
                chk = jnp.where(neg, -mag, mag)
                cev, cod = _widen(chk)
                oev, ood = _widen(
                    plsc.bitcast(rows_v[r0 + K, slw], jnp.bfloat16))
                res_pair = []
                for g, ch, ow in ((2 * m, cev, oev), (2 * m + 1, cod, ood)):
                    sl = pl.ds(g * LANES, LANES)
                    res = (io_v[j, sl] + wck * ch
                           + wr0 * ow + wr1 * prv_v[j, sl])
                    if final:
                        res = 1.0 / (1.0 + jnp.exp(-res))
                    io_v[j, sl] = res
                    res_pair.append(res)
                sout_v[j, slw] = _narrow(res_pair[0], res_pair[1])
            return _

        lax.fori_loop(0, NCH, node_body, None, unroll=2)

    def outer(c0, _):
        for b in range(2):
            c = c0 + b
            pltpu.make_async_copy(
                sbf.at[idx_v.at[c]], bufs[b][0], bufs[b][1]).wait()
            compute(c, bufs[b][0])

            @pl.when(c + 2 < CPT)
            def _issue_next():
                issue(c + 2, b)
        return _

    lax.fori_loop(0, CPT // 2, lambda s, x: outer(s * 2, x), None)
    pltpu.async_copy(io_v, out.at[pl.ds(base, NPT)], sem0)
    pltpu.async_copy(sout_v, sout.at[pl.ds(base, NPT)], sem1)
    pltpu.make_async_copy(io_v, out.at[pl.ds(base, NPT)], sem0).wait()
    pltpu.make_async_copy(sout_v, sout.at[pl.ds(base, NPT)], sem1).wait()


@functools.partial(jax.jit, static_argnames=("final",))
def _bp_iter(sbf, prv, wllr, w3, idx, final):
    mesh = plsc.VectorSubcoreMesh(core_axis_name="c", subcore_axis_name="s")
    return pl.kernel(
        functools.partial(_iter_body, final),
        out_type=(
            jax.ShapeDtypeStruct((N, B), jnp.float32),
            jax.ShapeDtypeStruct((N, B // 2), jnp.int32),
        ),
        mesh=mesh,
        compiler_params=pltpu.CompilerParams(
            needs_layout_passes=False, use_tc_tiling_on_sc=False),
        scratch_types=[
            pltpu.VMEM((CPT, IDXW), jnp.int32),
            pltpu.VMEM((IDXW, B // 2), jnp.int32),
            pltpu.VMEM((IDXW, B // 2), jnp.int32),
            pltpu.VMEM((NPT, B), jnp.float32),
            pltpu.VMEM((NPT, B), jnp.float32),
            pltpu.VMEM((NPT, B // 2), jnp.int32),
            pltpu.VMEM((NPT * 3 + LANES,), jnp.float32),
            pltpu.SemaphoreType.DMA,
            pltpu.SemaphoreType.DMA,
            pltpu.SemaphoreType.DMA,
            pltpu.SemaphoreType.DMA,
            pltpu.SemaphoreType.DMA,
        ],
    )(sbf, prv, wllr, w3, idx)


def _shadow_init(x_f32):
    n = x_f32.shape[0]
    inter = (x_f32.astype(jnp.bfloat16)
             .reshape(n, NM, 2, LANES)
             .transpose(0, 1, 3, 2)
             .reshape(n, B // 2, 2))
    return lax.bitcast_convert_type(inter, jnp.int32)


def kernel(input_llr, check_index_tensor, var_index_tensor, w_ch, w_check,
           w_res):
    del var_index_tensor
    llr_t = input_llr.T
    wllr_t = (input_llr * w_ch[None, :]).T
    own = jnp.arange(N, dtype=jnp.int32)[:, None]
    idx = jnp.concatenate(
        [check_index_tensor.astype(jnp.int32), own], axis=1)
    idx = idx.reshape(NW, CPT, IDXW)
    zeros = jnp.zeros_like(w_res[0])
    sbf = _shadow_init(wllr_t)
    prv = wllr_t
    out = wllr_t
    for t in range(5):
        w3 = jnp.stack(
            [w_check, w_res[0], w_res[1] if t > 0 else zeros], axis=1)
        w3 = jnp.concatenate(
            [w3.reshape(-1), jnp.zeros((LANES,), jnp.float32)])
        wl = wllr_t + llr_t if t == 4 else wllr_t
        new, new_sbf = _bp_iter(sbf, prv, wl, w3, idx, final=(t == 4))
        prv, out, sbf = out, new, new_sbf
    return out.T
